# baseline (speedup 1.0000x reference)
import jax
import jax.numpy as jnp
from jax.experimental import pallas as pl


def _transpose_body(x_ref, o_ref):
    for g in range(3):
        o_ref[0, g, :, :, :85] = jnp.transpose(x_ref[0, g * 85:(g + 1) * 85], (1, 2, 0))


def kernel(input):
    bs, ch, in_h, in_w = input.shape
    attrs = 85
    groups = ch // attrs
    out = pl.pallas_call(
        _transpose_body,
        grid=(bs,),
        in_specs=[pl.BlockSpec((1, ch, in_h, in_w), lambda b: (b, 0, 0, 0))],
        out_specs=pl.BlockSpec((1, groups, in_h, in_w, 128), lambda b: (b, 0, 0, 0, 0)),
        out_shape=jax.ShapeDtypeStruct((bs, groups, in_h, in_w, 128), jnp.float32),
    )(input)
    return out[..., :attrs]


# speedup vs baseline: 1.1024x; 1.1024x over previous
import jax
import jax.numpy as jnp
from jax.experimental import pallas as pl


def _transpose_body(x_ref, o_ref):
    for g in range(3):
        o_ref[0, g] = jnp.transpose(x_ref[0, g * 85:(g + 1) * 85], (1, 2, 0))


def kernel(input):
    bs, ch, in_h, in_w = input.shape
    attrs = 85
    groups = ch // attrs
    out = pl.pallas_call(
        _transpose_body,
        grid=(bs,),
        in_specs=[pl.BlockSpec((1, ch, in_h, in_w), lambda b: (b, 0, 0, 0))],
        out_specs=pl.BlockSpec((1, groups, in_h, in_w, attrs), lambda b: (b, 0, 0, 0, 0)),
        out_shape=jax.ShapeDtypeStruct((bs, groups, in_h, in_w, attrs), jnp.float32),
    )(input)
    return out


# speedup vs baseline: 1.1028x; 1.0004x over previous
import jax
import jax.numpy as jnp
from jax.experimental import pallas as pl


def _transpose_body(x0_ref, x1_ref, x2_ref, o_ref):
    for g, ref in enumerate((x0_ref, x1_ref, x2_ref)):
        o_ref[0, g] = jnp.transpose(ref[0], (1, 2, 0))


def kernel(input):
    bs, ch, in_h, in_w = input.shape
    attrs = 85
    groups = ch // attrs
    out = pl.pallas_call(
        _transpose_body,
        grid=(bs,),
        in_specs=[
            pl.BlockSpec((1, attrs, in_h, in_w),
                         lambda b, _g=g: (b, _g, 0, 0))
            for g in range(groups)
        ],
        out_specs=pl.BlockSpec((1, groups, in_h, in_w, attrs), lambda b: (b, 0, 0, 0, 0)),
        out_shape=jax.ShapeDtypeStruct((bs, groups, in_h, in_w, attrs), jnp.float32),
    )(input, input, input)
    return out
